# baseline (speedup 1.0000x reference)
import functools

import jax
import jax.numpy as jnp
from jax import lax
from jax.experimental import pallas as pl
from jax.experimental.pallas import tpu as pltpu
from jax.experimental.pallas import tpu_sc as plsc

T = 2048
H = 768
I = 1536
E = 8
K = 2
BTILE = 1024
NT_R = T * K // BTILE + E
P = NT_R * BTILE
NW = 32
CHUNK = T // NW
REP = 128
BTSH = 512


def _router_kernel(x_ref, wg_ref, w1r_ref, w2r_ref, p1_ref,
                   p2_ref, et_ref, valid_ref, aux_ref):
    x = x_ref[...]
    wg = wg_ref[...]
    logits = lax.dot_general(
        x, wg, (((1,), (1,)), ((), ())), preferred_element_type=jnp.float32)
    m = jnp.max(logits, axis=-1, keepdims=True)
    ex = jnp.exp(logits - m)
    denom = jnp.sum(ex, axis=-1, keepdims=True)
    probs = ex / denom

    ii = lax.broadcasted_iota(jnp.int32, (E, E), 0)
    jj = lax.broadcasted_iota(jnp.int32, (E, E), 1)
    strict_e = (ii < jj).astype(jnp.float32)
    m1 = jnp.max(probs, axis=-1, keepdims=True)
    is1 = probs >= m1
    prior1 = jnp.dot(is1.astype(jnp.float32), strict_e,
                     preferred_element_type=jnp.float32)
    first = is1 & (prior1 == 0.0)
    probs2 = jnp.where(first, -jnp.inf, probs)
    m2 = jnp.max(probs2, axis=-1, keepdims=True)
    is2 = probs2 >= m2
    prior2 = jnp.dot(is2.astype(jnp.float32), strict_e,
                     preferred_element_type=jnp.float32)
    second = is2 & (prior2 == 0.0)
    rden = 1.0 / (m1 + m2 + 1e-9)
    w1 = m1 * rden
    w2 = m2 * rden
    w1r_ref[...] = jnp.broadcast_to(w1, (T, REP))
    w2r_ref[...] = jnp.broadcast_to(w2, (T, REP))

    firstf = first.astype(jnp.float32)
    secondf = second.astype(jnp.float32)
    c = firstf + secondf

    ti = lax.broadcasted_iota(jnp.int32, (T, T), 0)
    tj = lax.broadcasted_iota(jnp.int32, (T, T), 1)
    strict_t = (tj < ti).astype(jnp.bfloat16)
    rank = lax.dot_general(strict_t, c.astype(jnp.bfloat16),
                           (((1,), (0,)), ((), ())),
                           preferred_element_type=jnp.float32)

    counts = jnp.sum(c, axis=0, keepdims=True)
    pc = ((counts.astype(jnp.int32) + BTILE - 1) >> 10) << 10
    pc_f = pc.astype(jnp.float32)
    off_f = jnp.dot(pc_f, strict_e, preferred_element_type=jnp.float32)
    ends_f = off_f + pc_f
    slot = off_f + rank
    p1_ref[...] = jnp.sum(firstf * slot, axis=1).astype(jnp.int32)
    p2_ref[...] = jnp.sum(secondf * slot, axis=1).astype(jnp.int32)

    js = lax.broadcasted_iota(jnp.int32, (NT_R, E), 0).astype(
        jnp.float32) * float(BTILE)
    et_ref[...] = jnp.sum((js >= ends_f).astype(jnp.float32),
                          axis=1).astype(jnp.int32)
    total = ends_f[0:1, E - 1:E]
    valid_ref[...] = (js[:, 0:1] < total).astype(jnp.int32)[:, 0]

    tokens_per_expert = counts[0] / (T * K + 1e-9)
    avg_probs = jnp.mean(probs, axis=0)
    load_balance = E * jnp.sum(tokens_per_expert * avg_probs)
    lse = jnp.log(denom[:, 0]) + m[:, 0]
    z_loss = jnp.mean(jnp.square(lse)) * 0.001
    entropy = jnp.mean(-jnp.sum(probs * jnp.log(probs + 1e-9), axis=-1))
    entropy_loss = (jnp.log(jnp.float32(E)) - entropy) * 0.01
    usage = jnp.mean((tokens_per_expert > 0.01).astype(jnp.float32))
    utilization_loss = (1.0 - usage) * 0.1
    total_loss = load_balance + z_loss + entropy_loss + utilization_loss
    aux_ref[...] = total_loss[None, None]


def _dispatch_body(x_hbm, p1_hbm, p2_hbm, w1r_hbm, w2r_hbm,
                   xs_hbm, wslot_hbm,
                   xrow_v, idx1_v, idx2_v, w1_v, w2_v, sem1, sem2):
    wid = lax.axis_index("s") * 2 + lax.axis_index("c")
    base = wid * CHUNK
    pltpu.sync_copy(p1_hbm.at[pl.ds(base, CHUNK)], idx1_v)
    pltpu.sync_copy(p2_hbm.at[pl.ds(base, CHUNK)], idx2_v)
    pltpu.sync_copy(x_hbm.at[pl.ds(base, CHUNK)], xrow_v)
    pltpu.sync_copy(w1r_hbm.at[pl.ds(base, CHUNK)], w1_v)
    pltpu.sync_copy(w2r_hbm.at[pl.ds(base, CHUNK)], w2_v)
    cp1 = pltpu.async_copy(xrow_v, xs_hbm.at[idx1_v], sem1)
    cp2 = pltpu.async_copy(w1_v, wslot_hbm.at[idx1_v], sem2)
    cp1.wait()
    cp3 = pltpu.async_copy(xrow_v, xs_hbm.at[idx2_v], sem1)
    cp2.wait()
    cp4 = pltpu.async_copy(w2_v, wslot_hbm.at[idx2_v], sem2)
    cp3.wait()
    cp4.wait()


def _ffn(x, wg, wu, wd):
    dn = (((1,), (1,)), ((), ()))
    gate = lax.dot_general(x, wg, dn, preferred_element_type=jnp.float32)
    up = lax.dot_general(x, wu, dn, preferred_element_type=jnp.float32)
    h = (gate * lax.logistic(gate) * up).astype(jnp.bfloat16)
    return lax.dot_general(h, wd, dn, preferred_element_type=jnp.float32)


def _grouped_kernel(et_sref, valid_sref, xs_ref, wg_ref, wu_ref, wd_ref,
                    wslot_ref, ys_ref):
    j = pl.program_id(0)

    @pl.when(valid_sref[j] == 1)
    def _():
        o = _ffn(xs_ref[...].astype(jnp.bfloat16),
                 wg_ref[0].astype(jnp.bfloat16),
                 wu_ref[0].astype(jnp.bfloat16),
                 wd_ref[0].astype(jnp.bfloat16))
        ys_ref[...] = o * wslot_ref[:, 0:1]


def _shared_kernel(x_ref, wsg_ref, wsu_ref, wsd_ref, sgate_ref, ysh_ref):
    i = pl.program_id(0)

    @pl.when(i == 0)
    def _():
        ysh_ref[...] = jnp.zeros_like(ysh_ref)

    o = _ffn(x_ref[...].astype(jnp.bfloat16),
             wsg_ref[...].astype(jnp.bfloat16),
             wsu_ref[...].astype(jnp.bfloat16),
             wsd_ref[...].astype(jnp.bfloat16))
    ysh_ref[...] += o * lax.logistic(sgate_ref[...])


def _gather_body(ys_hbm, p1_hbm, p2_hbm, g1_hbm, g2_hbm,
                 r1_v, r2_v, idx1_v, idx2_v, sem1, sem2):
    wid = lax.axis_index("s") * 2 + lax.axis_index("c")
    base = wid * CHUNK
    pltpu.sync_copy(p1_hbm.at[pl.ds(base, CHUNK)], idx1_v)
    pltpu.sync_copy(p2_hbm.at[pl.ds(base, CHUNK)], idx2_v)
    cp1 = pltpu.async_copy(ys_hbm.at[idx1_v], r1_v, sem1)
    cp2 = pltpu.async_copy(ys_hbm.at[idx2_v], r2_v, sem2)
    cp1.wait()
    pltpu.sync_copy(r1_v, g1_hbm.at[pl.ds(base, CHUNK)])
    cp2.wait()
    pltpu.sync_copy(r2_v, g2_hbm.at[pl.ds(base, CHUNK)])


def _final_add_kernel(g1_ref, g2_ref, ysh_ref, out_ref):
    out_ref[...] = (g1_ref[...] + g2_ref[...] +
                    ysh_ref[...].astype(jnp.float32))


def kernel(hidden_states, W_gate, Wg, Wu, Wd, Wsg, Wsu, Wsd, shared_gate):
    b, s, h = hidden_states.shape
    x = hidden_states.reshape(-1, h)

    (w1r, w2r, p1c, p2c, et, valid, aux) = pl.pallas_call(
        _router_kernel,
        out_shape=(
            jax.ShapeDtypeStruct((T, REP), jnp.float32),
            jax.ShapeDtypeStruct((T, REP), jnp.float32),
            jax.ShapeDtypeStruct((T,), jnp.int32),
            jax.ShapeDtypeStruct((T,), jnp.int32),
            jax.ShapeDtypeStruct((NT_R,), jnp.int32),
            jax.ShapeDtypeStruct((NT_R,), jnp.int32),
            jax.ShapeDtypeStruct((1, 1), jnp.float32),
        ),
    )(x, W_gate)

    pos1 = p1c
    pos2 = p2c

    mesh = plsc.VectorSubcoreMesh(core_axis_name="c", subcore_axis_name="s")
    dispatch = pl.kernel(
        _dispatch_body,
        mesh=mesh,
        out_type=(
            jax.ShapeDtypeStruct((P, H), jnp.float32),
            jax.ShapeDtypeStruct((P, REP), jnp.float32),
        ),
        scratch_types=[
            pltpu.VMEM((CHUNK, H), jnp.float32),
            pltpu.VMEM((CHUNK,), jnp.int32),
            pltpu.VMEM((CHUNK,), jnp.int32),
            pltpu.VMEM((CHUNK, REP), jnp.float32),
            pltpu.VMEM((CHUNK, REP), jnp.float32),
            pltpu.SemaphoreType.DMA,
            pltpu.SemaphoreType.DMA,
        ],
    )
    xs, wslot = dispatch(x, pos1, pos2, w1r, w2r)

    BISH = I // 3
    ysh = pl.pallas_call(
        _shared_kernel,
        grid=(3,),
        in_specs=[
            pl.BlockSpec((T, H), lambda i: (0, 0)),
            pl.BlockSpec((BISH, H), lambda i: (i, 0)),
            pl.BlockSpec((BISH, H), lambda i: (i, 0)),
            pl.BlockSpec((H, BISH), lambda i: (0, i)),
            pl.BlockSpec((1, 1), lambda i: (0, 0)),
        ],
        out_specs=pl.BlockSpec((T, H), lambda i: (0, 0)),
        out_shape=jax.ShapeDtypeStruct((T, H), jnp.float32),
    )(x, Wsg, Wsu, Wsd, shared_gate.reshape(1, 1))

    grid_spec = pltpu.PrefetchScalarGridSpec(
        num_scalar_prefetch=2,
        grid=(NT_R,),
        in_specs=[
            pl.BlockSpec((BTILE, H), lambda j, et_s, v_s: (j, 0)),
            pl.BlockSpec((1, I, H),
                         lambda j, et_s, v_s: (jnp.minimum(et_s[j], E - 1), 0, 0)),
            pl.BlockSpec((1, I, H),
                         lambda j, et_s, v_s: (jnp.minimum(et_s[j], E - 1), 0, 0)),
            pl.BlockSpec((1, H, I),
                         lambda j, et_s, v_s: (jnp.minimum(et_s[j], E - 1), 0, 0)),
            pl.BlockSpec((BTILE, REP), lambda j, et_s, v_s: (j, 0)),
        ],
        out_specs=pl.BlockSpec((BTILE, H), lambda j, et_s, v_s: (j, 0)),
    )
    ys = pl.pallas_call(
        _grouped_kernel,
        grid_spec=grid_spec,
        out_shape=jax.ShapeDtypeStruct((P, H), jnp.float32),
    )(et, valid, xs, Wg, Wu, Wd, wslot)

    gather = pl.kernel(
        _gather_body,
        mesh=mesh,
        out_type=(
            jax.ShapeDtypeStruct((T, H), jnp.float32),
            jax.ShapeDtypeStruct((T, H), jnp.float32),
        ),
        scratch_types=[
            pltpu.VMEM((CHUNK, H), jnp.float32),
            pltpu.VMEM((CHUNK, H), jnp.float32),
            pltpu.VMEM((CHUNK,), jnp.int32),
            pltpu.VMEM((CHUNK,), jnp.int32),
            pltpu.SemaphoreType.DMA,
            pltpu.SemaphoreType.DMA,
        ],
    )
    g1, g2 = gather(ys, pos1, pos2)

    BTA = 512
    out = pl.pallas_call(
        _final_add_kernel,
        grid=(T // BTA,),
        in_specs=[
            pl.BlockSpec((BTA, H), lambda t: (t, 0)),
            pl.BlockSpec((BTA, H), lambda t: (t, 0)),
            pl.BlockSpec((BTA, H), lambda t: (t, 0)),
        ],
        out_specs=pl.BlockSpec((BTA, H), lambda t: (t, 0)),
        out_shape=jax.ShapeDtypeStruct((T, H), jnp.float32),
    )(g1, g2, ysh)

    return out.reshape(b, s, h), aux[0, 0]


# speedup vs baseline: 1.0552x; 1.0552x over previous
import functools

import jax
import jax.numpy as jnp
from jax import lax
from jax.experimental import pallas as pl
from jax.experimental.pallas import tpu as pltpu
from jax.experimental.pallas import tpu_sc as plsc

T = 2048
H = 768
I = 1536
E = 8
K = 2
BTILE = 512
NT_R = T * K // BTILE + E
P = NT_R * BTILE
NW = 32
CHUNK = T // NW
REP = 128
BTSH = 512


def _router_kernel(x_ref, wg_ref, w1r_ref, w2r_ref, p1_ref,
                   p2_ref, et_ref, valid_ref, aux_ref):
    x = x_ref[...]
    wg = wg_ref[...]
    logits = lax.dot_general(
        x, wg, (((1,), (1,)), ((), ())), preferred_element_type=jnp.float32)
    m = jnp.max(logits, axis=-1, keepdims=True)
    ex = jnp.exp(logits - m)
    denom = jnp.sum(ex, axis=-1, keepdims=True)
    probs = ex / denom

    ii = lax.broadcasted_iota(jnp.int32, (E, E), 0)
    jj = lax.broadcasted_iota(jnp.int32, (E, E), 1)
    strict_e = (ii < jj).astype(jnp.float32)
    m1 = jnp.max(probs, axis=-1, keepdims=True)
    is1 = probs >= m1
    prior1 = jnp.dot(is1.astype(jnp.float32), strict_e,
                     preferred_element_type=jnp.float32)
    first = is1 & (prior1 == 0.0)
    probs2 = jnp.where(first, -jnp.inf, probs)
    m2 = jnp.max(probs2, axis=-1, keepdims=True)
    is2 = probs2 >= m2
    prior2 = jnp.dot(is2.astype(jnp.float32), strict_e,
                     preferred_element_type=jnp.float32)
    second = is2 & (prior2 == 0.0)
    rden = 1.0 / (m1 + m2 + 1e-9)
    w1 = m1 * rden
    w2 = m2 * rden
    w1r_ref[...] = w1
    w2r_ref[...] = w2

    firstf = first.astype(jnp.float32)
    secondf = second.astype(jnp.float32)
    c = firstf + secondf

    ti = lax.broadcasted_iota(jnp.int32, (T, T), 0)
    tj = lax.broadcasted_iota(jnp.int32, (T, T), 1)
    strict_t = (tj < ti).astype(jnp.bfloat16)
    rank = lax.dot_general(strict_t, c.astype(jnp.bfloat16),
                           (((1,), (0,)), ((), ())),
                           preferred_element_type=jnp.float32)

    counts = jnp.sum(c, axis=0, keepdims=True)
    pc = ((counts.astype(jnp.int32) + BTILE - 1) >> 9) << 9
    pc_f = pc.astype(jnp.float32)
    off_f = jnp.dot(pc_f, strict_e, preferred_element_type=jnp.float32)
    ends_f = off_f + pc_f
    slot = off_f + rank
    p1_ref[...] = jnp.sum(firstf * slot, axis=1).astype(jnp.int32)
    p2_ref[...] = jnp.sum(secondf * slot, axis=1).astype(jnp.int32)

    js = lax.broadcasted_iota(jnp.int32, (NT_R, E), 0).astype(
        jnp.float32) * float(BTILE)
    et_ref[...] = jnp.sum((js >= ends_f).astype(jnp.float32),
                          axis=1).astype(jnp.int32)
    total = ends_f[0:1, E - 1:E]
    valid_ref[...] = (js[:, 0:1] < total).astype(jnp.int32)[:, 0]

    tokens_per_expert = counts[0] / (T * K + 1e-9)
    avg_probs = jnp.mean(probs, axis=0)
    load_balance = E * jnp.sum(tokens_per_expert * avg_probs)
    lse = jnp.log(denom[:, 0]) + m[:, 0]
    z_loss = jnp.mean(jnp.square(lse)) * 0.001
    entropy = jnp.mean(-jnp.sum(probs * jnp.log(probs + 1e-9), axis=-1))
    entropy_loss = (jnp.log(jnp.float32(E)) - entropy) * 0.01
    usage = jnp.mean((tokens_per_expert > 0.01).astype(jnp.float32))
    utilization_loss = (1.0 - usage) * 0.1
    total_loss = load_balance + z_loss + entropy_loss + utilization_loss
    aux_ref[...] = total_loss[None, None]


def _dispatch_body(x_hbm, p1_hbm, p2_hbm, xs_hbm,
                   xrow_v, idx1_v, idx2_v, sem1):
    wid = lax.axis_index("s") * 2 + lax.axis_index("c")
    base = wid * CHUNK
    pltpu.sync_copy(p1_hbm.at[pl.ds(base, CHUNK)], idx1_v)
    pltpu.sync_copy(p2_hbm.at[pl.ds(base, CHUNK)], idx2_v)
    pltpu.sync_copy(x_hbm.at[pl.ds(base, CHUNK)], xrow_v)
    cp1 = pltpu.async_copy(xrow_v, xs_hbm.at[idx1_v], sem1)
    cp1.wait()
    cp3 = pltpu.async_copy(xrow_v, xs_hbm.at[idx2_v], sem1)
    cp3.wait()


def _ffn(x, wg, wu, wd):
    dn = (((1,), (1,)), ((), ()))
    gate = lax.dot_general(x, wg, dn, preferred_element_type=jnp.float32)
    up = lax.dot_general(x, wu, dn, preferred_element_type=jnp.float32)
    h = (gate * lax.logistic(gate) * up).astype(jnp.bfloat16)
    return lax.dot_general(h, wd, dn, preferred_element_type=jnp.float32)


def _grouped_kernel(et_sref, valid_sref, xs_ref, wg_ref, wu_ref, wd_ref,
                    ys_ref):
    j = pl.program_id(0)

    @pl.when(valid_sref[j] == 1)
    def _():
        ys_ref[...] = _ffn(xs_ref[...].astype(jnp.bfloat16),
                           wg_ref[0].astype(jnp.bfloat16),
                           wu_ref[0].astype(jnp.bfloat16),
                           wd_ref[0].astype(jnp.bfloat16))


def _shared_kernel(x_ref, wsg_ref, wsu_ref, wsd_ref, sgate_ref, ysh_ref):
    i = pl.program_id(0)

    @pl.when(i == 0)
    def _():
        ysh_ref[...] = jnp.zeros_like(ysh_ref)

    o = _ffn(x_ref[...].astype(jnp.bfloat16),
             wsg_ref[...].astype(jnp.bfloat16),
             wsu_ref[...].astype(jnp.bfloat16),
             wsd_ref[...].astype(jnp.bfloat16))
    ysh_ref[...] += o * lax.logistic(sgate_ref[...])


def _gather_body(ys_hbm, p1_hbm, p2_hbm, g1_hbm, g2_hbm,
                 r1_v, r2_v, idx1_v, idx2_v, sem1, sem2):
    wid = lax.axis_index("s") * 2 + lax.axis_index("c")
    base = wid * CHUNK
    pltpu.sync_copy(p1_hbm.at[pl.ds(base, CHUNK)], idx1_v)
    pltpu.sync_copy(p2_hbm.at[pl.ds(base, CHUNK)], idx2_v)
    cp1 = pltpu.async_copy(ys_hbm.at[idx1_v], r1_v, sem1)
    cp2 = pltpu.async_copy(ys_hbm.at[idx2_v], r2_v, sem2)
    cp1.wait()
    pltpu.sync_copy(r1_v, g1_hbm.at[pl.ds(base, CHUNK)])
    cp2.wait()
    pltpu.sync_copy(r2_v, g2_hbm.at[pl.ds(base, CHUNK)])


def _final_add_kernel(g1_ref, g2_ref, ysh_ref, w1_ref, w2_ref, out_ref):
    out_ref[...] = (w1_ref[...] * g1_ref[...] + w2_ref[...] * g2_ref[...] +
                    ysh_ref[...].astype(jnp.float32))


def kernel(hidden_states, W_gate, Wg, Wu, Wd, Wsg, Wsu, Wsd, shared_gate):
    b, s, h = hidden_states.shape
    x = hidden_states.reshape(-1, h)

    (w1r, w2r, p1c, p2c, et, valid, aux) = pl.pallas_call(
        _router_kernel,
        out_shape=(
            jax.ShapeDtypeStruct((T, 1), jnp.float32),
            jax.ShapeDtypeStruct((T, 1), jnp.float32),
            jax.ShapeDtypeStruct((T,), jnp.int32),
            jax.ShapeDtypeStruct((T,), jnp.int32),
            jax.ShapeDtypeStruct((NT_R,), jnp.int32),
            jax.ShapeDtypeStruct((NT_R,), jnp.int32),
            jax.ShapeDtypeStruct((1, 1), jnp.float32),
        ),
    )(x, W_gate)

    pos1 = p1c
    pos2 = p2c

    mesh = plsc.VectorSubcoreMesh(core_axis_name="c", subcore_axis_name="s")
    dispatch = pl.kernel(
        _dispatch_body,
        mesh=mesh,
        out_type=jax.ShapeDtypeStruct((P, H), jnp.float32),
        scratch_types=[
            pltpu.VMEM((CHUNK, H), jnp.float32),
            pltpu.VMEM((CHUNK,), jnp.int32),
            pltpu.VMEM((CHUNK,), jnp.int32),
            pltpu.SemaphoreType.DMA,
        ],
    )
    xs = dispatch(x, pos1, pos2)

    BISH = I // 3
    ysh = pl.pallas_call(
        _shared_kernel,
        grid=(3,),
        in_specs=[
            pl.BlockSpec((T, H), lambda i: (0, 0)),
            pl.BlockSpec((BISH, H), lambda i: (i, 0)),
            pl.BlockSpec((BISH, H), lambda i: (i, 0)),
            pl.BlockSpec((H, BISH), lambda i: (0, i)),
            pl.BlockSpec((1, 1), lambda i: (0, 0)),
        ],
        out_specs=pl.BlockSpec((T, H), lambda i: (0, 0)),
        out_shape=jax.ShapeDtypeStruct((T, H), jnp.float32),
    )(x, Wsg, Wsu, Wsd, shared_gate.reshape(1, 1))

    grid_spec = pltpu.PrefetchScalarGridSpec(
        num_scalar_prefetch=2,
        grid=(NT_R,),
        in_specs=[
            pl.BlockSpec((BTILE, H), lambda j, et_s, v_s: (j, 0)),
            pl.BlockSpec((1, I, H),
                         lambda j, et_s, v_s: (jnp.minimum(et_s[j], E - 1), 0, 0)),
            pl.BlockSpec((1, I, H),
                         lambda j, et_s, v_s: (jnp.minimum(et_s[j], E - 1), 0, 0)),
            pl.BlockSpec((1, H, I),
                         lambda j, et_s, v_s: (jnp.minimum(et_s[j], E - 1), 0, 0)),
        ],
        out_specs=pl.BlockSpec((BTILE, H), lambda j, et_s, v_s: (j, 0)),
    )
    ys = pl.pallas_call(
        _grouped_kernel,
        grid_spec=grid_spec,
        out_shape=jax.ShapeDtypeStruct((P, H), jnp.float32),
    )(et, valid, xs, Wg, Wu, Wd)

    gather = pl.kernel(
        _gather_body,
        mesh=mesh,
        out_type=(
            jax.ShapeDtypeStruct((T, H), jnp.float32),
            jax.ShapeDtypeStruct((T, H), jnp.float32),
        ),
        scratch_types=[
            pltpu.VMEM((CHUNK, H), jnp.float32),
            pltpu.VMEM((CHUNK, H), jnp.float32),
            pltpu.VMEM((CHUNK,), jnp.int32),
            pltpu.VMEM((CHUNK,), jnp.int32),
            pltpu.SemaphoreType.DMA,
            pltpu.SemaphoreType.DMA,
        ],
    )
    g1, g2 = gather(ys, pos1, pos2)

    BTA = 512
    out = pl.pallas_call(
        _final_add_kernel,
        grid=(T // BTA,),
        in_specs=[
            pl.BlockSpec((BTA, H), lambda t: (t, 0)),
            pl.BlockSpec((BTA, H), lambda t: (t, 0)),
            pl.BlockSpec((BTA, H), lambda t: (t, 0)),
            pl.BlockSpec((BTA, 1), lambda t: (t, 0)),
            pl.BlockSpec((BTA, 1), lambda t: (t, 0)),
        ],
        out_specs=pl.BlockSpec((BTA, H), lambda t: (t, 0)),
        out_shape=jax.ShapeDtypeStruct((T, H), jnp.float32),
    )(g1, g2, ysh, w1r, w2r)

    return out.reshape(b, s, h), aux[0, 0]
